# Initial kernel scaffold; baseline (speedup 1.0000x reference)
import jax
import jax.numpy as jnp
from jax.experimental import pallas as pl


def kernel(x, edge_index, W0, b0, W1, a1s, a1d, b1, W2, a2s, a2d, b2, W3, b3):
    raise NotImplementedError("write your pallas kernel here")


# speedup vs baseline: 12.1304x; 12.1304x over previous
import functools

import jax
import jax.numpy as jnp
from jax import lax
from jax.experimental import pallas as pl
from jax.experimental.pallas import tpu as pltpu
from jax.experimental.pallas import tpu_sc as plsc

N = 50000
E = 800000
NODE_DIM = 64
HID = 128
OUT = 256
HEADS = 4

ROWS = 1000
GRID = N // ROWS
NEG = -100.0

C1, ROUNDS1, B1 = 176, 9, 128
C2, ROUNDS2, B2 = 528, 3, 128
NTILES = 32


def _k1_body(x_ref, w0_ref, b0_ref, vs_ref, vd_ref, h0_ref, as_ref, ad_ref,
             mx_ref):
    i = pl.program_id(0)
    h0 = jnp.maximum(jnp.dot(x_ref[...], w0_ref[...],
                             preferred_element_type=jnp.float32)
                     + b0_ref[...], 0.0)
    h0_ref[...] = h0
    ts = jnp.dot(h0, vs_ref[...], preferred_element_type=jnp.float32)
    td = jnp.dot(h0, vd_ref[...], preferred_element_type=jnp.float32)
    col = lax.broadcasted_iota(jnp.int32, (ROWS, 16), 1)
    as_ref[...] = jnp.where(col < HEADS, ts, NEG)
    ad_ref[...] = jnp.where(col < HEADS, td, NEG)
    mxs = jnp.max(ts, axis=0, keepdims=True)
    mxd = jnp.max(td, axis=0, keepdims=True)
    upd = jnp.concatenate([mxs, mxd, jnp.full((6, 16), -1e30)], axis=0)

    @pl.when(i == 0)
    def _():
        mx_ref[...] = jnp.full((8, 16), -1e30)

    mx_ref[...] = jnp.maximum(mx_ref[...], upd)


def _k1(x, w0, b0, vs, vd):
    return pl.pallas_call(
        _k1_body,
        grid=(GRID,),
        in_specs=[
            pl.BlockSpec((ROWS, NODE_DIM), lambda i: (i, 0)),
            pl.BlockSpec((NODE_DIM, HID), lambda i: (0, 0)),
            pl.BlockSpec((1, HID), lambda i: (0, 0)),
            pl.BlockSpec((HID, 16), lambda i: (0, 0)),
            pl.BlockSpec((HID, 16), lambda i: (0, 0)),
        ],
        out_specs=[
            pl.BlockSpec((ROWS, HID), lambda i: (i, 0)),
            pl.BlockSpec((ROWS, 16), lambda i: (i, 0)),
            pl.BlockSpec((ROWS, 16), lambda i: (i, 0)),
            pl.BlockSpec((8, 16), lambda i: (0, 0)),
        ],
        out_shape=[
            jax.ShapeDtypeStruct((N, HID), jnp.float32),
            jax.ShapeDtypeStruct((N, 16), jnp.float32),
            jax.ShapeDtypeStruct((N, 16), jnp.float32),
            jax.ShapeDtypeStruct((8, 16), jnp.float32),
        ],
    )(x, w0, b0, vs, vd)


def _k2_body(agg_ref, den_ref, w1_ref, b1_ref, w2_ref, a2s_ref, a2d_ref,
             h2_ref, as_ref, ad_ref, mx_ref):
    i = pl.program_id(0)
    agg = agg_ref[...]
    den = den_ref[...]
    outs = []
    for k in range(HEADS):
        cagg = agg[:, k * HID:(k + 1) * HID] / (den[:, k:k + 1] + 1e-16)
        outs.append(jnp.dot(cagg, w1_ref[:, k * HID:(k + 1) * HID],
                            preferred_element_type=jnp.float32))
    o1 = jnp.concatenate(outs, axis=1) + b1_ref[...]
    u = jnp.where(o1 > 0, o1, jnp.exp(jnp.minimum(o1, 0.0)) - 1.0)
    h2 = jnp.dot(u, w2_ref[...], preferred_element_type=jnp.float32)
    h2_ref[...] = h2
    ts = jnp.dot(h2, a2s_ref[...], preferred_element_type=jnp.float32)
    td = jnp.dot(h2, a2d_ref[...], preferred_element_type=jnp.float32)
    col = lax.broadcasted_iota(jnp.int32, (ROWS, 16), 1)
    as_ref[...] = jnp.where(col < 1, ts, NEG)
    ad_ref[...] = jnp.where(col < 1, td, NEG)
    mxs = jnp.max(ts, axis=0, keepdims=True)
    mxd = jnp.max(td, axis=0, keepdims=True)
    upd = jnp.concatenate([mxs, mxd, jnp.full((6, 16), -1e30)], axis=0)

    @pl.when(i == 0)
    def _():
        mx_ref[...] = jnp.full((8, 16), -1e30)

    mx_ref[...] = jnp.maximum(mx_ref[...], upd)


def _k2(agg1, den1, w1, b1, w2, a2s, a2d):
    return pl.pallas_call(
        _k2_body,
        grid=(GRID,),
        in_specs=[
            pl.BlockSpec((ROWS, HEADS * HID), lambda i: (i, 0)),
            pl.BlockSpec((ROWS, 16), lambda i: (i, 0)),
            pl.BlockSpec((HID, HEADS * HID), lambda i: (0, 0)),
            pl.BlockSpec((1, HEADS * HID), lambda i: (0, 0)),
            pl.BlockSpec((HEADS * HID, HID), lambda i: (0, 0)),
            pl.BlockSpec((HID, 16), lambda i: (0, 0)),
            pl.BlockSpec((HID, 16), lambda i: (0, 0)),
        ],
        out_specs=[
            pl.BlockSpec((ROWS, HID), lambda i: (i, 0)),
            pl.BlockSpec((ROWS, 16), lambda i: (i, 0)),
            pl.BlockSpec((ROWS, 16), lambda i: (i, 0)),
            pl.BlockSpec((8, 16), lambda i: (0, 0)),
        ],
        out_shape=[
            jax.ShapeDtypeStruct((N, HID), jnp.float32),
            jax.ShapeDtypeStruct((N, 16), jnp.float32),
            jax.ShapeDtypeStruct((N, 16), jnp.float32),
            jax.ShapeDtypeStruct((8, 16), jnp.float32),
        ],
    )(agg1, den1, w1, b1, w2, a2s, a2d)


def _k3_body(agg_ref, den_ref, b2_ref, w3_ref, out_ref, acc_ref):
    i = pl.program_id(0)
    v = agg_ref[...] / (den_ref[:, 0:1] + 1e-16) + b2_ref[...]
    v = jnp.where(v > 0, v, jnp.exp(jnp.minimum(v, 0.0)) - 1.0)
    s = jnp.sum(v, axis=0, keepdims=True)

    @pl.when(i == 0)
    def _():
        acc_ref[...] = jnp.zeros((8, HID), jnp.float32)

    acc_ref[0:1, :] = acc_ref[0:1, :] + s

    @pl.when(i == GRID - 1)
    def _():
        g = acc_ref[0:1, :] * (1.0 / N)
        out_ref[...] = jnp.dot(g, w3_ref[...],
                               preferred_element_type=jnp.float32)


def _k3(agg2, den2, b2, w3):
    return pl.pallas_call(
        _k3_body,
        grid=(GRID,),
        in_specs=[
            pl.BlockSpec((ROWS, HID), lambda i: (i, 0)),
            pl.BlockSpec((ROWS, 16), lambda i: (i, 0)),
            pl.BlockSpec((1, HID), lambda i: (0, 0)),
            pl.BlockSpec((HID, OUT), lambda i: (0, 0)),
        ],
        out_specs=pl.BlockSpec((1, OUT), lambda i: (0, 0)),
        out_shape=jax.ShapeDtypeStruct((1, OUT), jnp.float32),
        scratch_shapes=[pltpu.VMEM((8, HID), jnp.float32)],
    )(agg2, den2, b2, w3)


def _sc_edge(feat, asa, ada, srcs, dsts, off, m16, heads, c_rows, rounds, bat):
    np_rows = rounds * NTILES * c_rows
    rw = heads * HID
    nspans = rounds * NTILES
    noff = ((nspans + 1 + 7) // 8) * 8
    mesh = plsc.VectorSubcoreMesh(core_axis_name="c", subcore_axis_name="s",
                                  num_cores=2, num_subcores=16)

    @functools.partial(
        pl.kernel,
        out_type=[
            jax.ShapeDtypeStruct((np_rows, rw), jnp.float32),
            jax.ShapeDtypeStruct((np_rows, 16), jnp.float32),
        ],
        mesh=mesh,
        compiler_params=pltpu.CompilerParams(use_tc_tiling_on_sc=False),
        scratch_types=[
            pltpu.VMEM((c_rows, rw), jnp.float32),
            pltpu.VMEM((c_rows, 16), jnp.float32),
            pltpu.VMEM((bat,), jnp.int32),
            pltpu.VMEM((bat,), jnp.int32),
            pltpu.VMEM((bat + 16,), jnp.int32),
            pltpu.VMEM((bat, HID), jnp.float32),
            pltpu.VMEM((bat, 16), jnp.float32),
            pltpu.VMEM((bat, 16), jnp.float32),
            pltpu.VMEM((noff + 16,), jnp.int32),
            pltpu.VMEM((16,), jnp.float32),
            pltpu.SemaphoreType.DMA,
        ],
    )
    def k(feat_h, asa_h, ada_h, srcs_h, dsts_h, off_h, m_h, agg_o, den_o,
          acc, den, sidx, didxg, didx, fbuf, abuf, bbuf, offv, mbuf, sem):
        wid = lax.axis_index("s") * 2 + lax.axis_index("c")
        pltpu.sync_copy(off_h, offv.at[pl.ds(0, noff)])
        pltpu.sync_copy(m_h, mbuf)
        mvec = mbuf[...]

        def round_body(r, _):
            span = r * NTILES + wid
            base = span * c_rows

            def zrow(i, _):
                for q in range(rw // 16):
                    acc[i, pl.ds(q * 16, 16)] = jnp.zeros((16,), jnp.float32)
                den[i] = jnp.zeros((16,), jnp.float32)
                return 0

            lax.fori_loop(0, c_rows, zrow, 0)

            ov = offv[pl.ds(span, 16)]
            e0 = ov[0]
            e1 = ov[1]
            ebase = e0 - lax.rem(e0, 8)
            nbat = (e1 - ebase + (bat - 1)) // bat

            def batch_body(b, _):
                eb = pl.multiple_of(ebase + b * bat, 8)
                pltpu.sync_copy(srcs_h.at[pl.ds(eb, bat)], sidx)
                pltpu.sync_copy(dsts_h.at[pl.ds(eb, bat)], didxg)
                pltpu.sync_copy(dsts_h.at[pl.ds(eb, bat)],
                                didx.at[pl.ds(0, bat)])
                pltpu.async_copy(feat_h.at[sidx], fbuf, sem).wait()
                pltpu.async_copy(asa_h.at[sidx], abuf, sem).wait()
                pltpu.async_copy(ada_h.at[didxg], bbuf, sem).wait()

                def edge_body(j, _):
                    ea = eb + j

                    @pl.when((ea >= e0) & (ea < e1))
                    def _():
                        z = abuf[j] + bbuf[j]
                        z = jnp.where(z > 0, z, z * 0.2)
                        w = jnp.exp(z - mvec)
                        ld = didx[pl.ds(j, 16)][0] - base
                        den[ld] = den[ld] + w
                        for kk in range(heads):
                            wk = w[kk]
                            for q in range(HID // 16):
                                sl = pl.ds(kk * HID + q * 16, 16)
                                fsl = pl.ds(q * 16, 16)
                                acc[ld, sl] = acc[ld, sl] + wk * fbuf[j, fsl]

                    return 0

                lax.fori_loop(0, bat, edge_body, 0)
                return 0

            lax.fori_loop(0, nbat, batch_body, 0)
            pltpu.sync_copy(acc, agg_o.at[pl.ds(base, c_rows)])
            pltpu.sync_copy(den, den_o.at[pl.ds(base, c_rows)])
            return 0

        lax.fori_loop(0, rounds, round_body, 0)

    return k(feat, asa, ada, srcs, dsts, off, m16)


def kernel(x, edge_index, W0, b0, W1, a1s, a1d, b1, W2, a2s, a2d, b2, W3, b3):
    src = edge_index[0].astype(jnp.uint32)
    dst = edge_index[1].astype(jnp.uint32)
    packed = jnp.sort(dst * jnp.uint32(65536) + src)
    dsts = (packed // jnp.uint32(65536)).astype(jnp.int32)
    srcs = (packed % jnp.uint32(65536)).astype(jnp.int32)
    epad = ((E + B1 + 7) // 8) * 8
    srcs_p = jnp.pad(srcs, (0, epad - E))
    dsts_p = jnp.pad(dsts, (0, epad - E), constant_values=N)

    def spans(c_rows, rounds):
        ns = rounds * NTILES
        noff = ((ns + 1 + 7) // 8) * 8
        bounds = jnp.arange(noff, dtype=jnp.int32) * c_rows
        off = jnp.searchsorted(dsts, bounds.astype(jnp.int32),
                               side="left").astype(jnp.int32)
        return jnp.minimum(off, E)

    off1 = spans(C1, ROUNDS1)
    off2 = spans(C2, ROUNDS2)

    w1r = W1.reshape(HID, HEADS, HID)
    vs = jnp.einsum("dkc,kc->dk", w1r, a1s[0])
    vd = jnp.einsum("dkc,kc->dk", w1r, a1d[0])
    vs16 = jnp.pad(vs, ((0, 0), (0, 16 - HEADS)))
    vd16 = jnp.pad(vd, ((0, 0), (0, 16 - HEADS)))
    a2s16 = jnp.pad(a2s[0, 0][:, None], ((0, 0), (0, 15)))
    a2d16 = jnp.pad(a2d[0, 0][:, None], ((0, 0), (0, 15)))

    h0, as1, ad1, mx1 = _k1(x, W0, b0.reshape(1, HID), vs16, vd16)
    m1 = mx1[0, :] + mx1[1, :]
    m1 = jnp.where(m1 > 0, m1, m1 * 0.2)
    m16_1 = jnp.where(jnp.arange(16) < HEADS, m1, 0.0).astype(jnp.float32)

    agg1, den1 = _sc_edge(h0, as1, ad1, srcs_p, dsts_p, off1, m16_1,
                          HEADS, C1, ROUNDS1, B1)

    h2, as2, ad2, mx2 = _k2(agg1, den1, W1, b1.reshape(1, HEADS * HID), W2,
                            a2s16, a2d16)
    m2 = mx2[0, :] + mx2[1, :]
    m2 = jnp.where(m2 > 0, m2, m2 * 0.2)
    m16_2 = jnp.where(jnp.arange(16) < 1, m2, 0.0).astype(jnp.float32)

    agg2, den2 = _sc_edge(h2, as2, ad2, srcs_p, dsts_p, off2, m16_2,
                          1, C2, ROUNDS2, B2)

    return _k3(agg2, den2, b2.reshape(1, HID), W3)


# speedup vs baseline: 27.6877x; 2.2825x over previous
import functools

import jax
import jax.numpy as jnp
from jax import lax
from jax.experimental import pallas as pl
from jax.experimental.pallas import tpu as pltpu
from jax.experimental.pallas import tpu_sc as plsc

N = 50000
E = 800000
NODE_DIM = 64
HID = 128
OUT = 256
HEADS = 4

ROWS = 1000
GRID = N // ROWS
NEG = -100.0

C1, ROUNDS1, B1 = 176, 9, 128
C2, ROUNDS2, B2 = 528, 3, 128
NTILES = 32


def _k1_body(x_ref, w0_ref, b0_ref, vs_ref, vd_ref, h0_ref, as_ref, ad_ref,
             mx_ref):
    i = pl.program_id(0)
    h0 = jnp.maximum(jnp.dot(x_ref[...], w0_ref[...],
                             preferred_element_type=jnp.float32)
                     + b0_ref[...], 0.0)
    h0_ref[...] = h0
    ts = jnp.dot(h0, vs_ref[...], preferred_element_type=jnp.float32)
    td = jnp.dot(h0, vd_ref[...], preferred_element_type=jnp.float32)
    col = lax.broadcasted_iota(jnp.int32, (ROWS, 16), 1)
    as_ref[...] = jnp.where(col < HEADS, ts, NEG)
    ad_ref[...] = jnp.where(col < HEADS, td, NEG)
    mxs = jnp.max(ts, axis=0, keepdims=True)
    mxd = jnp.max(td, axis=0, keepdims=True)
    upd = jnp.concatenate([mxs, mxd, jnp.full((6, 16), -1e30)], axis=0)

    @pl.when(i == 0)
    def _():
        mx_ref[...] = jnp.full((8, 16), -1e30)

    mx_ref[...] = jnp.maximum(mx_ref[...], upd)


def _k1(x, w0, b0, vs, vd):
    return pl.pallas_call(
        _k1_body,
        grid=(GRID,),
        in_specs=[
            pl.BlockSpec((ROWS, NODE_DIM), lambda i: (i, 0)),
            pl.BlockSpec((NODE_DIM, HID), lambda i: (0, 0)),
            pl.BlockSpec((1, HID), lambda i: (0, 0)),
            pl.BlockSpec((HID, 16), lambda i: (0, 0)),
            pl.BlockSpec((HID, 16), lambda i: (0, 0)),
        ],
        out_specs=[
            pl.BlockSpec((ROWS, HID), lambda i: (i, 0)),
            pl.BlockSpec((ROWS, 16), lambda i: (i, 0)),
            pl.BlockSpec((ROWS, 16), lambda i: (i, 0)),
            pl.BlockSpec((8, 16), lambda i: (0, 0)),
        ],
        out_shape=[
            jax.ShapeDtypeStruct((N, HID), jnp.float32),
            jax.ShapeDtypeStruct((N, 16), jnp.float32),
            jax.ShapeDtypeStruct((N, 16), jnp.float32),
            jax.ShapeDtypeStruct((8, 16), jnp.float32),
        ],
    )(x, w0, b0, vs, vd)


def _k2_body(agg_ref, den_ref, w1_ref, b1_ref, w2_ref, a2s_ref, a2d_ref,
             h2_ref, as_ref, ad_ref, mx_ref):
    i = pl.program_id(0)
    agg = agg_ref[...]
    den = den_ref[...]
    outs = []
    for k in range(HEADS):
        cagg = agg[:, k * HID:(k + 1) * HID] / (den[:, k:k + 1] + 1e-16)
        outs.append(jnp.dot(cagg, w1_ref[:, k * HID:(k + 1) * HID],
                            preferred_element_type=jnp.float32))
    o1 = jnp.concatenate(outs, axis=1) + b1_ref[...]
    u = jnp.where(o1 > 0, o1, jnp.exp(jnp.minimum(o1, 0.0)) - 1.0)
    h2 = jnp.dot(u, w2_ref[...], preferred_element_type=jnp.float32)
    h2_ref[...] = h2
    ts = jnp.dot(h2, a2s_ref[...], preferred_element_type=jnp.float32)
    td = jnp.dot(h2, a2d_ref[...], preferred_element_type=jnp.float32)
    col = lax.broadcasted_iota(jnp.int32, (ROWS, 16), 1)
    as_ref[...] = jnp.where(col < 1, ts, NEG)
    ad_ref[...] = jnp.where(col < 1, td, NEG)
    mxs = jnp.max(ts, axis=0, keepdims=True)
    mxd = jnp.max(td, axis=0, keepdims=True)
    upd = jnp.concatenate([mxs, mxd, jnp.full((6, 16), -1e30)], axis=0)

    @pl.when(i == 0)
    def _():
        mx_ref[...] = jnp.full((8, 16), -1e30)

    mx_ref[...] = jnp.maximum(mx_ref[...], upd)


def _k2(agg1, den1, w1, b1, w2, a2s, a2d):
    return pl.pallas_call(
        _k2_body,
        grid=(GRID,),
        in_specs=[
            pl.BlockSpec((ROWS, HEADS * HID), lambda i: (i, 0)),
            pl.BlockSpec((ROWS, 16), lambda i: (i, 0)),
            pl.BlockSpec((HID, HEADS * HID), lambda i: (0, 0)),
            pl.BlockSpec((1, HEADS * HID), lambda i: (0, 0)),
            pl.BlockSpec((HEADS * HID, HID), lambda i: (0, 0)),
            pl.BlockSpec((HID, 16), lambda i: (0, 0)),
            pl.BlockSpec((HID, 16), lambda i: (0, 0)),
        ],
        out_specs=[
            pl.BlockSpec((ROWS, HID), lambda i: (i, 0)),
            pl.BlockSpec((ROWS, 16), lambda i: (i, 0)),
            pl.BlockSpec((ROWS, 16), lambda i: (i, 0)),
            pl.BlockSpec((8, 16), lambda i: (0, 0)),
        ],
        out_shape=[
            jax.ShapeDtypeStruct((N, HID), jnp.float32),
            jax.ShapeDtypeStruct((N, 16), jnp.float32),
            jax.ShapeDtypeStruct((N, 16), jnp.float32),
            jax.ShapeDtypeStruct((8, 16), jnp.float32),
        ],
    )(agg1, den1, w1, b1, w2, a2s, a2d)


def _k3_body(agg_ref, den_ref, b2_ref, w3_ref, out_ref, acc_ref):
    i = pl.program_id(0)
    v = agg_ref[...] / (den_ref[:, 0:1] + 1e-16) + b2_ref[...]
    v = jnp.where(v > 0, v, jnp.exp(jnp.minimum(v, 0.0)) - 1.0)
    s = jnp.sum(v, axis=0, keepdims=True)

    @pl.when(i == 0)
    def _():
        acc_ref[...] = jnp.zeros((8, HID), jnp.float32)

    acc_ref[0:1, :] = acc_ref[0:1, :] + s

    @pl.when(i == GRID - 1)
    def _():
        g = acc_ref[0:1, :] * (1.0 / N)
        out_ref[...] = jnp.dot(g, w3_ref[...],
                               preferred_element_type=jnp.float32)


def _k3(agg2, den2, b2, w3):
    return pl.pallas_call(
        _k3_body,
        grid=(GRID,),
        in_specs=[
            pl.BlockSpec((ROWS, HID), lambda i: (i, 0)),
            pl.BlockSpec((ROWS, 16), lambda i: (i, 0)),
            pl.BlockSpec((1, HID), lambda i: (0, 0)),
            pl.BlockSpec((HID, OUT), lambda i: (0, 0)),
        ],
        out_specs=pl.BlockSpec((1, OUT), lambda i: (0, 0)),
        out_shape=jax.ShapeDtypeStruct((1, OUT), jnp.float32),
        scratch_shapes=[pltpu.VMEM((8, HID), jnp.float32)],
    )(agg2, den2, b2, w3)


def _sc_edge(feat, asa, ada, srcs, dsts, off, m16, heads, c_rows, rounds, bat):
    np_rows = rounds * NTILES * c_rows
    rw = heads * HID
    nspans = rounds * NTILES
    noff = ((nspans + 1 + 7) // 8) * 8
    mesh = plsc.VectorSubcoreMesh(core_axis_name="c", subcore_axis_name="s",
                                  num_cores=2, num_subcores=16)

    @functools.partial(
        pl.kernel,
        out_type=[
            jax.ShapeDtypeStruct((np_rows, rw), jnp.float32),
            jax.ShapeDtypeStruct((np_rows, 16), jnp.float32),
        ],
        mesh=mesh,
        compiler_params=pltpu.CompilerParams(use_tc_tiling_on_sc=False),
        scratch_types=[
            pltpu.VMEM((c_rows, rw), jnp.float32),
            pltpu.VMEM((c_rows, 16), jnp.float32),
            pltpu.VMEM((bat,), jnp.int32),
            pltpu.VMEM((bat,), jnp.int32),
            pltpu.VMEM((bat + 16,), jnp.int32),
            pltpu.VMEM((bat, HID), jnp.float32),
            pltpu.VMEM((bat, 16), jnp.float32),
            pltpu.VMEM((bat, 16), jnp.float32),
            pltpu.VMEM((noff + 16,), jnp.int32),
            pltpu.VMEM((16,), jnp.float32),
            pltpu.SemaphoreType.DMA,
        ],
    )
    def k(feat_h, asa_h, ada_h, srcs_h, dsts_h, off_h, m_h, agg_o, den_o,
          acc, den, sidx, didxg, didx, fbuf, abuf, bbuf, offv, mbuf, sem):
        wid = lax.axis_index("s") * 2 + lax.axis_index("c")
        pltpu.sync_copy(off_h, offv.at[pl.ds(0, noff)])
        pltpu.sync_copy(m_h, mbuf)
        mvec = mbuf[...]

        def round_body(r, _):
            span = r * NTILES + wid
            base = span * c_rows

            def zrow(i, _):
                for q in range(rw // 16):
                    acc[i, pl.ds(q * 16, 16)] = jnp.zeros((16,), jnp.float32)
                den[i] = jnp.zeros((16,), jnp.float32)
                return 0

            lax.fori_loop(0, c_rows, zrow, 0)

            ov = offv[pl.ds(span, 16)]
            e0 = ov[0]
            e1 = ov[1]
            ebase = e0 - lax.rem(e0, 8)
            nbat = (e1 - ebase + (bat - 1)) // bat

            def batch_body(b, _):
                eb = pl.multiple_of(ebase + b * bat, 8)
                pltpu.sync_copy(srcs_h.at[pl.ds(eb, bat)], sidx)
                pltpu.sync_copy(dsts_h.at[pl.ds(eb, bat)], didxg)
                pltpu.sync_copy(dsts_h.at[pl.ds(eb, bat)],
                                didx.at[pl.ds(0, bat)])
                cp_f = pltpu.async_copy(feat_h.at[sidx], fbuf, sem)
                cp_a = pltpu.async_copy(asa_h.at[sidx], abuf, sem)
                cp_b = pltpu.async_copy(ada_h.at[didxg], bbuf, sem)
                cp_f.wait()
                cp_a.wait()
                cp_b.wait()
                jstart = jnp.maximum(e0 - eb, 0)
                jend = jnp.maximum(jstart, jnp.minimum(bat, e1 - eb))

                def edge_body(j, _):
                    z = abuf[j] + bbuf[j]
                    z = jnp.where(z > 0, z, z * 0.2)
                    w = jnp.exp(z - mvec)
                    ld = didx[pl.ds(j, 16)][0] - base
                    plsc.addupdate(den.at[ld], w)
                    frow = [fbuf[j, pl.ds(q * 16, 16)]
                            for q in range(HID // 16)]
                    for kk in range(heads):
                        wk = w[kk]
                        for q in range(HID // 16):
                            sl = pl.ds(kk * HID + q * 16, 16)
                            plsc.addupdate(acc.at[ld, sl], wk * frow[q])
                    return 0

                lax.fori_loop(jstart, jend, edge_body, 0)
                return 0

            lax.fori_loop(0, nbat, batch_body, 0)
            pltpu.sync_copy(acc, agg_o.at[pl.ds(base, c_rows)])
            pltpu.sync_copy(den, den_o.at[pl.ds(base, c_rows)])
            return 0

        lax.fori_loop(0, rounds, round_body, 0)

    return k(feat, asa, ada, srcs, dsts, off, m16)


def kernel(x, edge_index, W0, b0, W1, a1s, a1d, b1, W2, a2s, a2d, b2, W3, b3):
    src = edge_index[0].astype(jnp.uint32)
    dst = edge_index[1].astype(jnp.uint32)
    packed = jnp.sort(dst * jnp.uint32(65536) + src)
    dsts = (packed // jnp.uint32(65536)).astype(jnp.int32)
    srcs = (packed % jnp.uint32(65536)).astype(jnp.int32)
    epad = ((E + B1 + 7) // 8) * 8
    srcs_p = jnp.pad(srcs, (0, epad - E))
    dsts_p = jnp.pad(dsts, (0, epad - E), constant_values=N)

    def spans(c_rows, rounds):
        ns = rounds * NTILES
        noff = ((ns + 1 + 7) // 8) * 8
        bounds = jnp.arange(noff, dtype=jnp.int32) * c_rows
        off = jnp.searchsorted(dsts, bounds.astype(jnp.int32),
                               side="left").astype(jnp.int32)
        return jnp.minimum(off, E)

    off1 = spans(C1, ROUNDS1)
    off2 = spans(C2, ROUNDS2)

    w1r = W1.reshape(HID, HEADS, HID)
    vs = jnp.einsum("dkc,kc->dk", w1r, a1s[0])
    vd = jnp.einsum("dkc,kc->dk", w1r, a1d[0])
    vs16 = jnp.pad(vs, ((0, 0), (0, 16 - HEADS)))
    vd16 = jnp.pad(vd, ((0, 0), (0, 16 - HEADS)))
    a2s16 = jnp.pad(a2s[0, 0][:, None], ((0, 0), (0, 15)))
    a2d16 = jnp.pad(a2d[0, 0][:, None], ((0, 0), (0, 15)))

    h0, as1, ad1, mx1 = _k1(x, W0, b0.reshape(1, HID), vs16, vd16)
    m1 = mx1[0, :] + mx1[1, :]
    m1 = jnp.where(m1 > 0, m1, m1 * 0.2)
    m16_1 = jnp.where(jnp.arange(16) < HEADS, m1, 0.0).astype(jnp.float32)

    agg1, den1 = _sc_edge(h0, as1, ad1, srcs_p, dsts_p, off1, m16_1,
                          HEADS, C1, ROUNDS1, B1)

    h2, as2, ad2, mx2 = _k2(agg1, den1, W1, b1.reshape(1, HEADS * HID), W2,
                            a2s16, a2d16)
    m2 = mx2[0, :] + mx2[1, :]
    m2 = jnp.where(m2 > 0, m2, m2 * 0.2)
    m16_2 = jnp.where(jnp.arange(16) < 1, m2, 0.0).astype(jnp.float32)

    agg2, den2 = _sc_edge(h2, as2, ad2, srcs_p, dsts_p, off2, m16_2,
                          1, C2, ROUNDS2, B2)

    return _k3(agg2, den2, b2.reshape(1, HID), W3)
